# baseline (speedup 1.0000x reference)
import functools

import jax
import jax.numpy as jnp
from jax.experimental import pallas as pl
from jax.experimental.pallas import tpu as pltpu

TOPK = 20
BLK = 9216
SUBW = 2304
PAD = 128


def _topk_kernel(q_ref, p_ref, vals_ref, idx_ref, rv, ri, sc_ref, *,
                 k_total, blk):
    i = pl.program_id(0)
    nb = pl.num_programs(0)
    qn = q_ref.shape[0]

    @pl.when(i == 0)
    def _init():
        rv[...] = jnp.full((qn, PAD), -jnp.inf, jnp.float32)
        ri[...] = jnp.zeros((qn, PAD), jnp.int32)

    q = q_ref[...]
    p = p_ref[...]
    scores = jax.lax.dot_general(
        q, p, (((1,), (1,)), ((), ())),
        preferred_element_type=jnp.float32,
    )
    lane = jax.lax.broadcasted_iota(jnp.int32, (qn, blk), 1)
    rem = k_total - i * blk
    scores = jnp.where(lane < rem, scores, -jnp.inf)
    l20 = jax.lax.broadcasted_iota(jnp.int32, (qn, PAD), 1)

    def th_of(rv_v):
        return jnp.max(jnp.where(l20 == TOPK - 1, rv_v, -jnp.inf),
                       axis=1, keepdims=True)

    def insert(rv_v, ri_v, m, idx_t, take):
        pos = jnp.sum(jnp.where(rv_v >= m, 1, 0), axis=1, keepdims=True)
        sv = jnp.roll(rv_v, 1, axis=1)
        si = jnp.roll(ri_v, 1, axis=1)
        nrv = jnp.where(l20 < pos, rv_v, jnp.where(l20 == pos, m, sv))
        nri = jnp.where(l20 < pos, ri_v, jnp.where(l20 == pos, idx_t, si))
        return (jnp.where(take, nrv, rv_v), jnp.where(take, nri, ri_v))

    def cond(c):
        rv_v, _ri_v, m = c
        return jnp.any(m > th_of(rv_v))

    def body(c):
        rv_v, ri_v, m = c
        sc = sc_ref[...]
        take = m > th_of(rv_v)
        sel = jnp.min(jnp.where(sc == m, lane, blk), axis=1, keepdims=True)
        hit = lane == sel
        rv2, ri2 = insert(rv_v, ri_v, m, i * blk + sel, take)
        sc2 = jnp.where(hit & take, -jnp.inf, sc)
        sc_ref[...] = sc2
        m2 = jnp.max(sc2, axis=1, keepdims=True)
        return rv2, ri2, m2

    @pl.when(i == 0)
    def _bootstrap():
        nv, ni = rv[...], ri[...]
        nsub = blk // SUBW
        lane_s = jax.lax.broadcasted_iota(jnp.int32, (qn, SUBW), 1)
        for s in range(nsub):
            sc0 = scores[:, s * SUBW:(s + 1) * SUBW]

            def scond(c):
                sc, rv_v, _ri_v, m = c
                return jnp.any(m > th_of(rv_v))

            def sbody(c, _base=s * SUBW):
                sc, rv_v, ri_v, m = c
                take = m > th_of(rv_v)
                sel = jnp.min(jnp.where(sc == m, lane_s, SUBW),
                              axis=1, keepdims=True)
                hit = lane_s == sel
                rv2, ri2 = insert(rv_v, ri_v, m, _base + sel, take)
                sc2 = jnp.where(hit & take, -jnp.inf, sc)
                m2 = jnp.max(sc2, axis=1, keepdims=True)
                return sc2, rv2, ri2, m2

            m0s = jnp.max(sc0, axis=1, keepdims=True)
            _, nv, ni, _ = jax.lax.while_loop(scond, sbody, (sc0, nv, ni, m0s))
        rv[...] = nv
        ri[...] = ni

    @pl.when(i != 0)
    def _merge():
        sc_ref[...] = scores
        m0 = jnp.max(scores, axis=1, keepdims=True)
        nv, ni, _ = jax.lax.while_loop(
            cond, body, (rv[...], ri[...], m0))
        rv[...] = nv
        ri[...] = ni

    @pl.when(i == nb - 1)
    def _out():
        vals_ref[...] = rv[...]
        idx_ref[...] = ri[...]


def _retrieve(q, p, blk):
    qn, d = q.shape
    k_total = p.shape[0]
    nb = pl.cdiv(k_total, blk)
    vals, idx = pl.pallas_call(
        functools.partial(_topk_kernel, k_total=k_total, blk=blk),
        grid=(nb,),
        in_specs=[
            pl.BlockSpec((qn, d), lambda i: (0, 0)),
            pl.BlockSpec((blk, d), lambda i: (i, 0)),
        ],
        out_specs=[
            pl.BlockSpec((qn, PAD), lambda i: (0, 0)),
            pl.BlockSpec((qn, PAD), lambda i: (0, 0)),
        ],
        out_shape=[
            jax.ShapeDtypeStruct((qn, PAD), jnp.float32),
            jax.ShapeDtypeStruct((qn, PAD), jnp.int32),
        ],
        scratch_shapes=[
            pltpu.VMEM((qn, PAD), jnp.float32),
            pltpu.VMEM((qn, PAD), jnp.int32),
            pltpu.VMEM((qn, blk), jnp.float32),
        ],
        compiler_params=pltpu.CompilerParams(
            dimension_semantics=("arbitrary",),
        ),
    )(q, p)
    return vals[:, :TOPK], idx[:, :TOPK]


def kernel(question_embeddings, passage_embeddings, topk):
    del topk
    logits, retrieved_indices = _retrieve(
        question_embeddings, passage_embeddings, BLK)
    return logits, retrieved_indices, question_embeddings


# speedup vs baseline: 1.0945x; 1.0945x over previous
import functools

import jax
import jax.numpy as jnp
from jax.experimental import pallas as pl
from jax.experimental.pallas import tpu as pltpu

TOPK = 20
BLK = 9216
PAD = 128


def _topk_kernel(q_ref, p_ref, vals_ref, idx_ref, rv, ri, sc_ref, *,
                 k_total, blk):
    i = pl.program_id(0)
    nb = pl.num_programs(0)
    qn = q_ref.shape[0]

    @pl.when(i == 0)
    def _init():
        rv[...] = jnp.full((qn, PAD), -jnp.inf, jnp.float32)
        ri[...] = jnp.zeros((qn, PAD), jnp.int32)

    q = q_ref[...]
    p = p_ref[...]
    scores = jax.lax.dot_general(
        q, p, (((1,), (1,)), ((), ())),
        preferred_element_type=jnp.float32,
    )
    lane = jax.lax.broadcasted_iota(jnp.int32, (qn, blk), 1)
    rem = k_total - i * blk
    scores = jnp.where(lane < rem, scores, -jnp.inf)
    l20 = jax.lax.broadcasted_iota(jnp.int32, (qn, PAD), 1)

    def th_of(rv_v):
        return jnp.max(jnp.where(l20 == TOPK - 1, rv_v, -jnp.inf),
                       axis=1, keepdims=True)

    def cond(c):
        rv_v, _ri_v, m = c
        return jnp.any(m > th_of(rv_v))

    def body(c):
        rv_v, ri_v, m = c
        sc = sc_ref[...]
        take = m > th_of(rv_v)
        sel = jnp.min(jnp.where(sc == m, lane, blk), axis=1, keepdims=True)
        hit = lane == sel
        idx_t = i * blk + sel
        pos = jnp.sum(jnp.where(rv_v >= m, 1, 0), axis=1, keepdims=True)
        sv = jnp.roll(rv_v, 1, axis=1)
        si = jnp.roll(ri_v, 1, axis=1)
        nrv = jnp.where(l20 < pos, rv_v, jnp.where(l20 == pos, m, sv))
        nri = jnp.where(l20 < pos, ri_v, jnp.where(l20 == pos, idx_t, si))
        rv2 = jnp.where(take, nrv, rv_v)
        ri2 = jnp.where(take, nri, ri_v)
        sc2 = jnp.where(hit & take, -jnp.inf, sc)
        sc_ref[...] = sc2
        m2 = jnp.max(sc2, axis=1, keepdims=True)
        return rv2, ri2, m2

    sc_ref[...] = scores
    m0 = jnp.max(scores, axis=1, keepdims=True)
    nv, ni, _ = jax.lax.while_loop(
        cond, body, (rv[...], ri[...], m0))
    rv[...] = nv
    ri[...] = ni

    @pl.when(i == nb - 1)
    def _out():
        vals_ref[...] = nv
        idx_ref[...] = ni


def _retrieve(q, p, blk):
    qn, d = q.shape
    k_total = p.shape[0]
    nb = pl.cdiv(k_total, blk)
    vals, idx = pl.pallas_call(
        functools.partial(_topk_kernel, k_total=k_total, blk=blk),
        grid=(nb,),
        in_specs=[
            pl.BlockSpec((qn, d), lambda i: (0, 0)),
            pl.BlockSpec((blk, d), lambda i: (i, 0)),
        ],
        out_specs=[
            pl.BlockSpec((qn, PAD), lambda i: (0, 0)),
            pl.BlockSpec((qn, PAD), lambda i: (0, 0)),
        ],
        out_shape=[
            jax.ShapeDtypeStruct((qn, PAD), jnp.float32),
            jax.ShapeDtypeStruct((qn, PAD), jnp.int32),
        ],
        scratch_shapes=[
            pltpu.VMEM((qn, PAD), jnp.float32),
            pltpu.VMEM((qn, PAD), jnp.int32),
            pltpu.VMEM((qn, blk), jnp.float32),
        ],
        compiler_params=pltpu.CompilerParams(
            dimension_semantics=("arbitrary",),
        ),
    )(q, p)
    return vals[:, :TOPK], idx[:, :TOPK]


def kernel(question_embeddings, passage_embeddings, topk):
    del topk
    logits, retrieved_indices = _retrieve(
        question_embeddings, passage_embeddings, BLK)
    return logits, retrieved_indices, question_embeddings
